# baseline (speedup 1.0000x reference)
import jax
import jax.numpy as jnp
from jax import lax
from jax.experimental import pallas as pl


B, N = 256, 64
EMB = 64
D_INV = 256
N_ATOM_TYPES = 100
N_CHARGE_TYPES = 7
N_BOND_TYPES = 8

BB = 16
JG = 8


def _kernel(at_ref, ach_ref, bt_ref, taT_ref, tcT_ref, bot_ref,
            w1T_ref, b1_ref, w2T_ref, b2_ref, invs_ref, edges_ref):
    f32 = jnp.float32
    bf16 = jnp.bfloat16
    TOK = BB * N

    at_all = at_ref[0]
    ach_all = ach_ref[0]
    oh_a = (jax.lax.broadcasted_iota(jnp.int32, (N_ATOM_TYPES, TOK), 0)
            == at_all).astype(f32)
    oh_c = (jax.lax.broadcasted_iota(jnp.int32, (N_CHARGE_TYPES, TOK), 0)
            == ach_all).astype(f32)
    xaT = jnp.dot(taT_ref[:, :], oh_a, preferred_element_type=f32)
    xcT = jnp.dot(tcT_ref[:, :], oh_c, preferred_element_type=f32)
    xT = jnp.concatenate([xaT, xcT], axis=0)
    hT = jnp.dot(w1T_ref[:, :], xT, preferred_element_type=f32) + b1_ref[:, :]
    hT = hT * jax.nn.sigmoid(hT)
    oT = jnp.dot(w2T_ref[:, :], hT, preferred_element_type=f32) + b2_ref[:, :]
    invs_ref[:, :, :] = oT.T.reshape(BB, N, D_INV)

    table = bot_ref[:, :]
    tbig = jnp.concatenate(
        [jnp.concatenate([table] * JG, axis=1)] * JG, axis=0)
    grow = jax.lax.broadcasted_iota(jnp.int32, (8 * JG, 64 * JG), 0) // 8
    gcol = jax.lax.broadcasted_iota(jnp.int32, (8 * JG, 64 * JG), 1) // 64
    G = jnp.where(grow == gcol, tbig, 0.0).astype(bf16)

    erow = jax.lax.broadcasted_iota(jnp.int32, (N, 8 * N), 0)
    ecol = jax.lax.broadcasted_iota(jnp.int32, (N, 8 * N), 1) // 8
    E8 = (erow == ecol).astype(bf16)

    btf = bt_ref[:, :, :].astype(bf16).reshape(BB * N, N)
    btrep = jnp.dot(btf, E8, preferred_element_type=f32)
    vmod = (jax.lax.broadcasted_iota(jnp.int32, (BB * N, 8 * N), 1)
            % 8).astype(f32)
    oh_all = (btrep == vmod).astype(bf16)

    for g in range(N // JG):
        oh_g = oh_all[:, g * 8 * JG:(g + 1) * 8 * JG]
        out_g = jnp.dot(oh_g, G, preferred_element_type=f32)
        edges_ref[:, :, g * 64 * JG:(g + 1) * 64 * JG] = (
            out_g.reshape(BB, N, 64 * JG))


def _impl(atom_types, bond_types, atom_mask, atom_charges,
          atom_type_table, charge_table, bond_table, W1, b1, W2, b2):
    del atom_mask
    at3 = atom_types.reshape(B // BB, 1, BB * N)
    ach3 = atom_charges.reshape(B // BB, 1, BB * N)

    grid = (B // BB,)
    full = lambda *shape: pl.BlockSpec(shape, lambda i: (0,) * len(shape))
    invs, edges = pl.pallas_call(
        _kernel,
        grid=grid,
        in_specs=[
            pl.BlockSpec((1, 1, BB * N), lambda i: (i, 0, 0)),
            pl.BlockSpec((1, 1, BB * N), lambda i: (i, 0, 0)),
            pl.BlockSpec((BB, N, N), lambda i: (i, 0, 0)),
            full(EMB, N_ATOM_TYPES),
            full(EMB, N_CHARGE_TYPES),
            full(N_BOND_TYPES, EMB),
            full(D_INV, 2 * EMB),
            full(D_INV, 1),
            full(D_INV, D_INV),
            full(D_INV, 1),
        ],
        out_specs=[
            pl.BlockSpec((BB, N, D_INV), lambda i: (i, 0, 0)),
            pl.BlockSpec((BB, N, N * EMB), lambda i: (i, 0, 0)),
        ],
        out_shape=[
            jax.ShapeDtypeStruct((B, N, D_INV), jnp.float32),
            jax.ShapeDtypeStruct((B, N, N * EMB), jnp.float32),
        ],
    )(at3, ach3, bond_types, atom_type_table.T, charge_table.T, bond_table,
      W1.T, b1.reshape(D_INV, 1), W2.T, b2.reshape(D_INV, 1))
    return invs, edges.reshape(B, N, N, EMB)


def _make_jitted():
    import jax.sharding as jsh
    from jax.experimental.layout import Format, Layout
    dev = jax.devices()[0]
    sd = jsh.SingleDeviceSharding(dev)
    lin4 = Format(Layout(major_to_minor=(0, 1, 2, 3), tiling=((64,),)), sd)
    _impl.__name__ = "kernel"
    return jax.jit(_impl, out_shardings=(Format(None, sd), lin4))


_jitted = None


def kernel(*args):
    global _jitted
    if _jitted is None:
        _jitted = _make_jitted()
    return _jitted(*args)


# speedup vs baseline: 1.2109x; 1.2109x over previous
import jax
import jax.numpy as jnp
from jax import lax
from jax.experimental import pallas as pl


B, N = 256, 64
EMB = 64
D_INV = 256
N_ATOM_TYPES = 100
N_CHARGE_TYPES = 7
N_BOND_TYPES = 8

BB = 16
JG = 8


def _kernel(at_ref, ach_ref, bt_ref, taT_ref, tcT_ref, bot_ref,
            w1T_ref, b1_ref, w2T_ref, b2_ref, invs_ref, edges_ref):
    f32 = jnp.float32
    bf16 = jnp.bfloat16
    TOK = BB * N

    at_all = at_ref[0]
    ach_all = ach_ref[0]
    oh_a = (jax.lax.broadcasted_iota(jnp.int32, (N_ATOM_TYPES, TOK), 0)
            == at_all).astype(f32)
    oh_c = (jax.lax.broadcasted_iota(jnp.int32, (N_CHARGE_TYPES, TOK), 0)
            == ach_all).astype(f32)
    xaT = jnp.dot(taT_ref[:, :], oh_a, preferred_element_type=f32)
    xcT = jnp.dot(tcT_ref[:, :], oh_c, preferred_element_type=f32)
    xT = jnp.concatenate([xaT, xcT], axis=0)
    hT = jnp.dot(w1T_ref[:, :], xT, preferred_element_type=f32) + b1_ref[:, :]
    hT = hT * jax.nn.sigmoid(hT)
    oT = jnp.dot(w2T_ref[:, :], hT, preferred_element_type=f32) + b2_ref[:, :]
    invs_ref[:, :, :] = oT.T.reshape(BB, N, D_INV)

    table = bot_ref[:, :]
    tbig = jnp.concatenate(
        [jnp.concatenate([table] * JG, axis=1)] * JG, axis=0)
    grow = jax.lax.broadcasted_iota(jnp.int32, (8 * JG, 64 * JG), 0) // 8
    gcol = jax.lax.broadcasted_iota(jnp.int32, (8 * JG, 64 * JG), 1) // 64
    G = jnp.where(grow == gcol, tbig, 0.0).astype(bf16)

    erow = jax.lax.broadcasted_iota(jnp.int32, (N, 8 * N), 0)
    ecol = jax.lax.broadcasted_iota(jnp.int32, (N, 8 * N), 1) // 8
    E8 = (erow == ecol).astype(bf16)

    btf = bt_ref[:, :, :].astype(bf16).reshape(BB * N, N)
    btrep = jnp.dot(btf, E8, preferred_element_type=f32)
    vmod = (jax.lax.broadcasted_iota(jnp.int32, (BB * N, 8 * N), 1)
            % 8).astype(f32)
    oh_all = (btrep == vmod).astype(bf16)

    for g in range(N // JG):
        oh_g = oh_all[:, g * 8 * JG:(g + 1) * 8 * JG]
        out_g = jnp.dot(oh_g, G, preferred_element_type=f32)
        edges_ref[:, :, g * 64 * JG:(g + 1) * 64 * JG] = (
            out_g.astype(bf16).reshape(BB, N, 64 * JG))


@jax.jit
def kernel(atom_types, bond_types, atom_mask, atom_charges,
           atom_type_table, charge_table, bond_table, W1, b1, W2, b2):
    del atom_mask
    at3 = atom_types.reshape(B // BB, 1, BB * N)
    ach3 = atom_charges.reshape(B // BB, 1, BB * N)

    grid = (B // BB,)
    full = lambda *shape: pl.BlockSpec(shape, lambda i: (0,) * len(shape))
    invs, edges = pl.pallas_call(
        _kernel,
        grid=grid,
        in_specs=[
            pl.BlockSpec((1, 1, BB * N), lambda i: (i, 0, 0)),
            pl.BlockSpec((1, 1, BB * N), lambda i: (i, 0, 0)),
            pl.BlockSpec((BB, N, N), lambda i: (i, 0, 0)),
            full(EMB, N_ATOM_TYPES),
            full(EMB, N_CHARGE_TYPES),
            full(N_BOND_TYPES, EMB),
            full(D_INV, 2 * EMB),
            full(D_INV, 1),
            full(D_INV, D_INV),
            full(D_INV, 1),
        ],
        out_specs=[
            pl.BlockSpec((BB, N, D_INV), lambda i: (i, 0, 0)),
            pl.BlockSpec((BB, N, N * EMB), lambda i: (i, 0, 0)),
        ],
        out_shape=[
            jax.ShapeDtypeStruct((B, N, D_INV), jnp.float32),
            jax.ShapeDtypeStruct((B, N, N * EMB), jnp.bfloat16),
        ],
    )(at3, ach3, bond_types, atom_type_table.T, charge_table.T, bond_table,
      W1.T, b1.reshape(D_INV, 1), W2.T, b2.reshape(D_INV, 1))
    return invs, edges.reshape(B, N, N, EMB).astype(jnp.float32)


# speedup vs baseline: 1.2165x; 1.0046x over previous
import jax
import jax.numpy as jnp
from jax import lax
from jax.experimental import pallas as pl


B, N = 256, 64
EMB = 64
D_INV = 256
N_ATOM_TYPES = 100
N_CHARGE_TYPES = 7
N_BOND_TYPES = 8

BB = 32
JG = 8


def _kernel(at_ref, ach_ref, bt_ref, taT_ref, tcT_ref, bot_ref,
            w1T_ref, b1_ref, w2T_ref, b2_ref, invs_ref, edges_ref):
    f32 = jnp.float32
    bf16 = jnp.bfloat16
    TOK = BB * N

    at_all = at_ref[0]
    ach_all = ach_ref[0]
    oh_a = (jax.lax.broadcasted_iota(jnp.int32, (N_ATOM_TYPES, TOK), 0)
            == at_all).astype(f32)
    oh_c = (jax.lax.broadcasted_iota(jnp.int32, (N_CHARGE_TYPES, TOK), 0)
            == ach_all).astype(f32)
    xaT = jnp.dot(taT_ref[:, :], oh_a, preferred_element_type=f32)
    xcT = jnp.dot(tcT_ref[:, :], oh_c, preferred_element_type=f32)
    xT = jnp.concatenate([xaT, xcT], axis=0)
    hT = jnp.dot(w1T_ref[:, :], xT, preferred_element_type=f32) + b1_ref[:, :]
    hT = hT * jax.nn.sigmoid(hT)
    oT = jnp.dot(w2T_ref[:, :], hT, preferred_element_type=f32) + b2_ref[:, :]
    invs_ref[:, :, :] = oT.T.reshape(BB, N, D_INV)

    table = bot_ref[:, :]
    tbig = jnp.concatenate(
        [jnp.concatenate([table] * JG, axis=1)] * JG, axis=0)
    grow = jax.lax.broadcasted_iota(jnp.int32, (8 * JG, 64 * JG), 0) // 8
    gcol = jax.lax.broadcasted_iota(jnp.int32, (8 * JG, 64 * JG), 1) // 64
    G = jnp.where(grow == gcol, tbig, 0.0).astype(bf16)

    erow = jax.lax.broadcasted_iota(jnp.int32, (N, 8 * N), 0)
    ecol = jax.lax.broadcasted_iota(jnp.int32, (N, 8 * N), 1) // 8
    E8 = (erow == ecol).astype(bf16)

    btf = bt_ref[:, :, :].astype(bf16).reshape(BB * N, N)
    btrep = jnp.dot(btf, E8, preferred_element_type=f32)
    vmod = (jax.lax.broadcasted_iota(jnp.int32, (BB * N, 8 * N), 1)
            % 8).astype(f32)
    oh_all = (btrep == vmod).astype(bf16)

    for g in range(N // JG):
        oh_g = oh_all[:, g * 8 * JG:(g + 1) * 8 * JG]
        out_g = jnp.dot(oh_g, G, preferred_element_type=f32)
        edges_ref[:, :, g * 64 * JG:(g + 1) * 64 * JG] = (
            out_g.astype(bf16).reshape(BB, N, 64 * JG))


@jax.jit
def kernel(atom_types, bond_types, atom_mask, atom_charges,
           atom_type_table, charge_table, bond_table, W1, b1, W2, b2):
    del atom_mask
    at3 = atom_types.reshape(B // BB, 1, BB * N)
    ach3 = atom_charges.reshape(B // BB, 1, BB * N)

    grid = (B // BB,)
    full = lambda *shape: pl.BlockSpec(shape, lambda i: (0,) * len(shape))
    invs, edges = pl.pallas_call(
        _kernel,
        grid=grid,
        in_specs=[
            pl.BlockSpec((1, 1, BB * N), lambda i: (i, 0, 0)),
            pl.BlockSpec((1, 1, BB * N), lambda i: (i, 0, 0)),
            pl.BlockSpec((BB, N, N), lambda i: (i, 0, 0)),
            full(EMB, N_ATOM_TYPES),
            full(EMB, N_CHARGE_TYPES),
            full(N_BOND_TYPES, EMB),
            full(D_INV, 2 * EMB),
            full(D_INV, 1),
            full(D_INV, D_INV),
            full(D_INV, 1),
        ],
        out_specs=[
            pl.BlockSpec((BB, N, D_INV), lambda i: (i, 0, 0)),
            pl.BlockSpec((BB, N, N * EMB), lambda i: (i, 0, 0)),
        ],
        out_shape=[
            jax.ShapeDtypeStruct((B, N, D_INV), jnp.float32),
            jax.ShapeDtypeStruct((B, N, N * EMB), jnp.bfloat16),
        ],
    )(at3, ach3, bond_types, atom_type_table.T, charge_table.T, bond_table,
      W1.T, b1.reshape(D_INV, 1), W2.T, b2.reshape(D_INV, 1))
    return invs, edges.reshape(B, N, N, EMB).astype(jnp.float32)
